# Initial kernel scaffold; baseline (speedup 1.0000x reference)
import jax
import jax.numpy as jnp
from jax.experimental import pallas as pl


def kernel(x, edge_index, edge_attr, W, b):
    raise NotImplementedError("write your pallas kernel here")


# speedup vs baseline: 7.9042x; 7.9042x over previous
import functools

import jax
import jax.numpy as jnp
from jax import lax
from jax.experimental import pallas as pl
from jax.experimental.pallas import tpu as pltpu
from jax.experimental.pallas import tpu_sc as plsc

N = 10000
E = 320000
D = 128

NC = 2
NS = 16
NW = NC * NS
BLK = 80
NBLK = 128
CHUNK = 32
NCHUNK = NBLK // CHUNK
E_PAD = NW * NBLK * BLK
N_PAD = 10240
RPT = N_PAD // NS

_mesh = plsc.VectorSubcoreMesh(core_axis_name="c", subcore_axis_name="s")


@functools.partial(
    pl.kernel,
    out_type=jax.ShapeDtypeStruct((NC, N_PAD), jnp.float32),
    mesh=_mesh,
    scratch_types=[
        pltpu.VMEM((CHUNK, BLK), jnp.int32),
        pltpu.VMEM((CHUNK, BLK), jnp.float32),
        pltpu.VMEM((RPT,), jnp.float32),
        pltpu.VMEM_SHARED((N_PAD,), jnp.float32),
    ],
)
def _deg_kernel(col_hbm, w_hbm, out_hbm, colb, wb, zb, acc):
    c = lax.axis_index("c")
    s = lax.axis_index("s")
    wid = c * NS + s

    def zfill(i, carry):
        zb[pl.ds(i * 16, 16)] = jnp.zeros((16,), jnp.float32)
        return carry

    lax.fori_loop(0, RPT // 16, zfill, 0)
    pltpu.sync_copy(zb, acc.at[pl.ds(s * RPT, RPT)])
    plsc.subcore_barrier()

    base = wid * NBLK

    def chunk_body(t, carry):
        row0 = base + t * CHUNK
        pltpu.sync_copy(col_hbm.at[pl.ds(row0, CHUNK)], colb)
        pltpu.sync_copy(w_hbm.at[pl.ds(row0, CHUNK)], wb)

        def blk_body(j, carry2):
            pltpu.sync_copy(wb.at[j], acc.at[colb.at[j]], add=True)
            return carry2

        lax.fori_loop(0, CHUNK, blk_body, 0)
        return carry

    lax.fori_loop(0, NCHUNK, chunk_body, 0)
    plsc.subcore_barrier()
    pltpu.sync_copy(acc.at[pl.ds(s * RPT, RPT)], out_hbm.at[c, pl.ds(s * RPT, RPT)])


@functools.partial(
    pl.kernel,
    out_type=jax.ShapeDtypeStruct((NC, N_PAD, D), jnp.float32),
    mesh=_mesh,
    scratch_types=[
        pltpu.VMEM((CHUNK, BLK), jnp.int32),
        pltpu.VMEM((CHUNK, BLK), jnp.int32),
        pltpu.VMEM((CHUNK, BLK), jnp.float32),
        pltpu.VMEM((BLK, D), jnp.float32),
        pltpu.VMEM((BLK, D), jnp.float32),
        pltpu.VMEM_SHARED((N_PAD, D), jnp.float32),
        pltpu.SemaphoreType.DMA,
    ],
)
def _prop_kernel(h_hbm, row_hbm, col_hbm, w_hbm, out_hbm,
                 rowb, colb, wb, rows, zb, acc, sem):
    c = lax.axis_index("c")
    s = lax.axis_index("s")
    wid = c * NS + s

    def zfill(r, carry):
        for i in range(8):
            zb[r, pl.ds(i * 16, 16)] = jnp.zeros((16,), jnp.float32)
        return carry

    lax.fori_loop(0, BLK, zfill, 0)
    for k in range(RPT // BLK):
        pltpu.sync_copy(zb, acc.at[pl.ds(s * RPT + k * BLK, BLK)])
    plsc.subcore_barrier()

    base = wid * NBLK

    def chunk_body(t, carry):
        row0 = base + t * CHUNK
        pltpu.sync_copy(row_hbm.at[pl.ds(row0, CHUNK)], rowb)
        pltpu.sync_copy(col_hbm.at[pl.ds(row0, CHUNK)], colb)
        pltpu.sync_copy(w_hbm.at[pl.ds(row0, CHUNK)], wb)

        def blk_body(j, carry2):
            pltpu.async_copy(h_hbm.at[rowb.at[j]], rows, sem).wait()

            def scale(g, carry3):
                wv = wb[j, pl.ds(g * 16, 16)]
                e0 = g * 16
                for l in range(16):
                    sv = wv[l]
                    for i in range(8):
                        rows[e0 + l, pl.ds(i * 16, 16)] = (
                            rows[e0 + l, pl.ds(i * 16, 16)] * sv)
                return carry3

            lax.fori_loop(0, BLK // 16, scale, 0)
            pltpu.sync_copy(rows, acc.at[colb.at[j]], add=True)
            return carry2

        lax.fori_loop(0, CHUNK, blk_body, 0)
        return carry

    lax.fori_loop(0, NCHUNK, chunk_body, 0)
    plsc.subcore_barrier()
    pltpu.sync_copy(acc.at[pl.ds(s * RPT, RPT)],
                    out_hbm.at[c, pl.ds(s * RPT, RPT)])


def _prep_body(x_ref, dp_ref, h0t_ref, dinv_ref):
    deg = 1.0 + dp_ref[:, 0:1] + dp_ref[:, 1:2]
    dinv = jnp.where(deg > 0, lax.rsqrt(deg), 0.0)
    h0t_ref[...] = x_ref[...] * dinv
    dinv_ref[...] = dinv


_TCB = 1000


def _prep_call(x, dp):
    return pl.pallas_call(
        _prep_body,
        grid=(N // _TCB,),
        in_specs=[
            pl.BlockSpec((_TCB, D), lambda i: (i, 0)),
            pl.BlockSpec((_TCB, 2), lambda i: (i, 0)),
        ],
        out_specs=[
            pl.BlockSpec((_TCB, D), lambda i: (i, 0)),
            pl.BlockSpec((_TCB, 1), lambda i: (i, 0)),
        ],
        out_shape=[
            jax.ShapeDtypeStruct((N, D), jnp.float32),
            jax.ShapeDtypeStruct((N, 1), jnp.float32),
        ],
    )(x, dp)


def _mid_body(p_ref, h0t_ref, dinv_ref, out_ref):
    t = p_ref[0] + p_ref[1] + h0t_ref[...]
    d = dinv_ref[...]
    out_ref[...] = t * (d * d)


def _mid_call(p, h0t, dinv):
    return pl.pallas_call(
        _mid_body,
        grid=(N // _TCB,),
        in_specs=[
            pl.BlockSpec((2, _TCB, D), lambda i: (0, i, 0)),
            pl.BlockSpec((_TCB, D), lambda i: (i, 0)),
            pl.BlockSpec((_TCB, 1), lambda i: (i, 0)),
        ],
        out_specs=pl.BlockSpec((_TCB, D), lambda i: (i, 0)),
        out_shape=jax.ShapeDtypeStruct((N, D), jnp.float32),
    )(p, h0t, dinv)


def _final_body(q_ref, h1t_ref, dinv_ref, w_ref, b_ref, out_ref):
    h2 = (q_ref[0] + q_ref[1] + h1t_ref[...]) * dinv_ref[...]
    z = jnp.dot(h2, w_ref[...], preferred_element_type=jnp.float32) + b_ref[...]
    z = jnp.maximum(z, 0.0)
    m = jnp.max(z, axis=-1, keepdims=True)
    lse = jnp.log(jnp.sum(jnp.exp(z - m), axis=-1, keepdims=True)) + m
    out_ref[...] = z - lse


def _final_call(q, h1t, dinv, w, b2):
    return pl.pallas_call(
        _final_body,
        grid=(N // _TCB,),
        in_specs=[
            pl.BlockSpec((2, _TCB, D), lambda i: (0, i, 0)),
            pl.BlockSpec((_TCB, D), lambda i: (i, 0)),
            pl.BlockSpec((_TCB, 1), lambda i: (i, 0)),
            pl.BlockSpec((D, D), lambda i: (0, 0)),
            pl.BlockSpec((1, D), lambda i: (0, 0)),
        ],
        out_specs=pl.BlockSpec((_TCB, D), lambda i: (i, 0)),
        out_shape=jax.ShapeDtypeStruct((N, D), jnp.float32),
    )(q, h1t, dinv, w, b2)


def kernel(x, edge_index, edge_attr, W, b):
    zpad_i = jnp.zeros((E_PAD - E,), jnp.int32)
    row = jnp.concatenate([edge_index[0], zpad_i]).reshape(E_PAD // BLK, BLK)
    col = jnp.concatenate([edge_index[1], zpad_i]).reshape(E_PAD // BLK, BLK)
    w2 = jnp.concatenate(
        [edge_attr, jnp.zeros((E_PAD - E,), jnp.float32)]).reshape(E_PAD // BLK, BLK)

    degp = _deg_kernel(col, w2)
    dp = jnp.transpose(degp[:, :N])
    h0t, dinv = _prep_call(x, dp)
    p = _prop_kernel(h0t, row, col, w2)
    h1t = _mid_call(p, h0t, dinv)
    q = _prop_kernel(h1t, row, col, w2)
    return _final_call(q, h1t, dinv, W, b.reshape(1, D))


# speedup vs baseline: 8.9219x; 1.1288x over previous
import functools

import jax
import jax.numpy as jnp
from jax import lax
from jax.experimental import pallas as pl
from jax.experimental.pallas import tpu as pltpu
from jax.experimental.pallas import tpu_sc as plsc

N = 10000
E = 320000
D = 128

NC = 2
NS = 16
NW = NC * NS
BLK = 128
NBLK = 80
E_PAD = NW * NBLK * BLK
N_PAD = 10240
RPT = N_PAD // NS
NBUF = 4

_mesh = plsc.VectorSubcoreMesh(core_axis_name="c", subcore_axis_name="s")


@functools.partial(
    pl.kernel,
    out_type=jax.ShapeDtypeStruct((NC, N_PAD), jnp.float32),
    mesh=_mesh,
    scratch_types=[
        pltpu.VMEM((NBLK, BLK), jnp.int32),
        pltpu.VMEM((NBLK, BLK), jnp.float32),
        pltpu.VMEM((RPT,), jnp.float32),
        pltpu.VMEM_SHARED((N_PAD,), jnp.float32),
    ],
)
def _deg_kernel(col_hbm, w_hbm, out_hbm, colb, wb, zb, acc):
    c = lax.axis_index("c")
    s = lax.axis_index("s")
    wid = c * NS + s

    def zfill(i, carry):
        zb[pl.ds(i * 16, 16)] = jnp.zeros((16,), jnp.float32)
        return carry

    lax.fori_loop(0, RPT // 16, zfill, 0)
    pltpu.sync_copy(zb, acc.at[pl.ds(s * RPT, RPT)])

    ebase = wid * NBLK
    pltpu.sync_copy(col_hbm.at[pl.ds(ebase, NBLK)], colb)
    pltpu.sync_copy(w_hbm.at[pl.ds(ebase, NBLK)], wb)
    plsc.subcore_barrier()

    def blk_body(j, carry):
        pltpu.sync_copy(wb.at[j], acc.at[colb.at[j]], add=True)
        return carry

    lax.fori_loop(0, NBLK, blk_body, 0)
    plsc.subcore_barrier()
    pltpu.sync_copy(acc.at[pl.ds(s * RPT, RPT)], out_hbm.at[c, pl.ds(s * RPT, RPT)])


CH = NBLK // 2


@functools.partial(
    pl.kernel,
    out_type=jax.ShapeDtypeStruct((NC, N_PAD, D), jnp.float32),
    mesh=_mesh,
    scratch_types=[
        pltpu.VMEM((CH, BLK), jnp.int32),
        pltpu.VMEM((CH, BLK), jnp.int32),
        pltpu.VMEM((CH, BLK), jnp.float32),
        pltpu.VMEM((BLK, D), jnp.float32),
        pltpu.VMEM((BLK, D), jnp.float32),
        pltpu.VMEM_SHARED((N_PAD, D), jnp.float32),
        pltpu.SemaphoreType.DMA,
        pltpu.SemaphoreType.DMA,
        pltpu.SemaphoreType.DMA,
        pltpu.SemaphoreType.DMA,
    ],
)
def _prop_kernel(h_hbm, row_hbm, col_hbm, w_hbm, out_hbm,
                 rowb, colb, wb, r0, r1, acc, g0, g1, s0, s1):
    c = lax.axis_index("c")
    s = lax.axis_index("s")
    wid = c * NS + s
    bufs = (r0, r1)
    gsem = (g0, g1)
    ssem = (s0, s1)

    def zfill(r, carry):
        for i in range(8):
            r0[r, pl.ds(i * 16, 16)] = jnp.zeros((16,), jnp.float32)
        return carry

    lax.fori_loop(0, BLK, zfill, 0)
    for k in range(RPT // BLK):
        pltpu.sync_copy(r0, acc.at[pl.ds(s * RPT + k * BLK, BLK)])
    plsc.subcore_barrier()

    ebase = wid * NBLK

    def issue_gather(k, p):
        pltpu.async_copy(h_hbm.at[rowb.at[k]], bufs[p], gsem[p])

    def wait_gather(k, p):
        pltpu.make_async_copy(h_hbm.at[rowb.at[k]], bufs[p], gsem[p]).wait()

    def issue_scatter(k, p):
        pltpu.async_copy(bufs[p], acc.at[colb.at[k]], ssem[p], add=True)

    def wait_scatter(k, p):
        pltpu.make_async_copy(bufs[p], acc.at[colb.at[k]], ssem[p]).wait()

    def scale_buf(buf, k):
        def scale(g, carry):
            wv = wb[k, pl.ds(g * 16, 16)]
            e0 = g * 16
            for l in range(16):
                sv = wv[l]
                for i in range(8):
                    buf[e0 + l, pl.ds(i * 16, 16)] = (
                        buf[e0 + l, pl.ds(i * 16, 16)] * sv)
            return carry

        lax.fori_loop(0, BLK // 16, scale, 0)

    for t in range(2):
        if t > 0:
            wait_scatter(CH - 1, (CH - 1) % 2)
        pltpu.sync_copy(row_hbm.at[pl.ds(ebase + t * CH, CH)], rowb)
        pltpu.sync_copy(col_hbm.at[pl.ds(ebase + t * CH, CH)], colb)
        pltpu.sync_copy(w_hbm.at[pl.ds(ebase + t * CH, CH)], wb)
        issue_gather(0, 0)
        issue_gather(1, 1)
        wait_gather(0, 0)
        scale_buf(bufs[0], 0)
        issue_scatter(0, 0)

        def pair(u, carry):
            k0 = 1 + 2 * u
            for par, k_off in ((1, 0), (0, 1)):
                k = k0 + k_off
                wait_scatter(k - 1, 1 - par)
                issue_gather(k + 1, 1 - par)
                wait_gather(k, par)
                scale_buf(bufs[par], k)
                issue_scatter(k, par)
            return carry

        lax.fori_loop(0, (CH - 2) // 2, pair, 0)

        k = CH - 1
        wait_scatter(k - 1, k % 2 ^ 1)
        wait_gather(k, k % 2)
        scale_buf(bufs[k % 2], k)
        issue_scatter(k, k % 2)

    wait_scatter(CH - 1, (CH - 1) % 2)
    plsc.subcore_barrier()
    pltpu.sync_copy(acc.at[pl.ds(s * RPT, RPT)],
                    out_hbm.at[c, pl.ds(s * RPT, RPT)])


def _prep_body(x_ref, dp_ref, h0t_ref, dinv_ref):
    deg = 1.0 + dp_ref[:, 0:1] + dp_ref[:, 1:2]
    dinv = jnp.where(deg > 0, lax.rsqrt(deg), 0.0)
    h0t_ref[...] = x_ref[...] * dinv
    dinv_ref[...] = dinv


_TCB = 1000


def _prep_call(x, dp):
    return pl.pallas_call(
        _prep_body,
        grid=(N // _TCB,),
        in_specs=[
            pl.BlockSpec((_TCB, D), lambda i: (i, 0)),
            pl.BlockSpec((_TCB, 2), lambda i: (i, 0)),
        ],
        out_specs=[
            pl.BlockSpec((_TCB, D), lambda i: (i, 0)),
            pl.BlockSpec((_TCB, 1), lambda i: (i, 0)),
        ],
        out_shape=[
            jax.ShapeDtypeStruct((N, D), jnp.float32),
            jax.ShapeDtypeStruct((N, 1), jnp.float32),
        ],
    )(x, dp)


def _mid_body(p_ref, h0t_ref, dinv_ref, out_ref):
    t = p_ref[0] + p_ref[1] + h0t_ref[...]
    d = dinv_ref[...]
    out_ref[...] = t * (d * d)


def _mid_call(p, h0t, dinv):
    return pl.pallas_call(
        _mid_body,
        grid=(N // _TCB,),
        in_specs=[
            pl.BlockSpec((2, _TCB, D), lambda i: (0, i, 0)),
            pl.BlockSpec((_TCB, D), lambda i: (i, 0)),
            pl.BlockSpec((_TCB, 1), lambda i: (i, 0)),
        ],
        out_specs=pl.BlockSpec((_TCB, D), lambda i: (i, 0)),
        out_shape=jax.ShapeDtypeStruct((N, D), jnp.float32),
    )(p, h0t, dinv)


def _final_body(q_ref, h1t_ref, dinv_ref, w_ref, b_ref, out_ref):
    h2 = (q_ref[0] + q_ref[1] + h1t_ref[...]) * dinv_ref[...]
    z = jnp.dot(h2, w_ref[...], preferred_element_type=jnp.float32) + b_ref[...]
    z = jnp.maximum(z, 0.0)
    m = jnp.max(z, axis=-1, keepdims=True)
    lse = jnp.log(jnp.sum(jnp.exp(z - m), axis=-1, keepdims=True)) + m
    out_ref[...] = z - lse


def _final_call(q, h1t, dinv, w, b2):
    return pl.pallas_call(
        _final_body,
        grid=(N // _TCB,),
        in_specs=[
            pl.BlockSpec((2, _TCB, D), lambda i: (0, i, 0)),
            pl.BlockSpec((_TCB, D), lambda i: (i, 0)),
            pl.BlockSpec((_TCB, 1), lambda i: (i, 0)),
            pl.BlockSpec((D, D), lambda i: (0, 0)),
            pl.BlockSpec((1, D), lambda i: (0, 0)),
        ],
        out_specs=pl.BlockSpec((_TCB, D), lambda i: (i, 0)),
        out_shape=jax.ShapeDtypeStruct((N, D), jnp.float32),
    )(q, h1t, dinv, w, b2)


def kernel(x, edge_index, edge_attr, W, b):
    zpad_i = jnp.zeros((E_PAD - E,), jnp.int32)
    row = jnp.concatenate([edge_index[0], zpad_i]).reshape(E_PAD // BLK, BLK)
    col = jnp.concatenate([edge_index[1], zpad_i]).reshape(E_PAD // BLK, BLK)
    w2 = jnp.concatenate(
        [edge_attr, jnp.zeros((E_PAD - E,), jnp.float32)]).reshape(E_PAD // BLK, BLK)

    degp = _deg_kernel(col, w2)
    dp = jnp.transpose(degp[:, :N])
    h0t, dinv = _prep_call(x, dp)
    p = _prop_kernel(h0t, row, col, w2)
    h1t = _mid_call(p, h0t, dinv)
    q = _prop_kernel(h1t, row, col, w2)
    return _final_call(q, h1t, dinv, W, b.reshape(1, D))


# speedup vs baseline: 27.4406x; 3.0756x over previous
import functools

import jax
import jax.numpy as jnp
from jax import lax
from jax.experimental import pallas as pl
from jax.experimental.pallas import tpu as pltpu
from jax.experimental.pallas import tpu_sc as plsc

N = 10000
E = 320000
D = 128

NC = 2
NS = 16
NW = NC * NS
BLK = 128
NBLK = 80
E_PAD = NW * NBLK * BLK
N_PAD = 10240
RPT = N_PAD // NS
NBUF = 4

_mesh = plsc.VectorSubcoreMesh(core_axis_name="c", subcore_axis_name="s")


@functools.partial(
    pl.kernel,
    out_type=jax.ShapeDtypeStruct((NC, N_PAD), jnp.float32),
    mesh=_mesh,
    scratch_types=[
        pltpu.VMEM((NBLK, BLK), jnp.int32),
        pltpu.VMEM((NBLK, BLK), jnp.float32),
        pltpu.VMEM((RPT,), jnp.float32),
        pltpu.VMEM_SHARED((N_PAD,), jnp.float32),
    ],
)
def _deg_kernel(col_hbm, w_hbm, out_hbm, colb, wb, zb, acc):
    c = lax.axis_index("c")
    s = lax.axis_index("s")
    wid = c * NS + s

    def zfill(i, carry):
        zb[pl.ds(i * 16, 16)] = jnp.zeros((16,), jnp.float32)
        return carry

    lax.fori_loop(0, RPT // 16, zfill, 0)
    pltpu.sync_copy(zb, acc.at[pl.ds(s * RPT, RPT)])

    ebase = wid * NBLK
    pltpu.sync_copy(col_hbm.at[pl.ds(ebase, NBLK)], colb)
    pltpu.sync_copy(w_hbm.at[pl.ds(ebase, NBLK)], wb)
    plsc.subcore_barrier()

    def blk_body(j, carry):
        pltpu.sync_copy(wb.at[j], acc.at[colb.at[j]], add=True)
        return carry

    lax.fori_loop(0, NBLK, blk_body, 0)
    plsc.subcore_barrier()
    pltpu.sync_copy(acc.at[pl.ds(s * RPT, RPT)], out_hbm.at[c, pl.ds(s * RPT, RPT)])


CH = NBLK // 2


@functools.partial(
    pl.kernel,
    out_type=jax.ShapeDtypeStruct((NC, N_PAD, D), jnp.float32),
    mesh=_mesh,
    scratch_types=[
        pltpu.VMEM((CH, BLK), jnp.int32),
        pltpu.VMEM((CH, BLK), jnp.int32),
        pltpu.VMEM((CH, BLK), jnp.float32),
        pltpu.VMEM((BLK, D), jnp.float32),
        pltpu.VMEM((BLK, D), jnp.float32),
        pltpu.VMEM_SHARED((N_PAD, D), jnp.float32),
        pltpu.SemaphoreType.DMA,
        pltpu.SemaphoreType.DMA,
        pltpu.SemaphoreType.DMA,
        pltpu.SemaphoreType.DMA,
    ],
)
def _prop_kernel(h_hbm, row_hbm, col_hbm, w_hbm, out_hbm,
                 rowb, colb, wb, r0, r1, acc, g0, g1, s0, s1):
    c = lax.axis_index("c")
    s = lax.axis_index("s")
    wid = c * NS + s
    bufs = (r0, r1)
    gsem = (g0, g1)
    ssem = (s0, s1)

    def zfill(r, carry):
        for i in range(8):
            r0[r, pl.ds(i * 16, 16)] = jnp.zeros((16,), jnp.float32)
        return carry

    lax.fori_loop(0, BLK, zfill, 0)
    for k in range(RPT // BLK):
        pltpu.sync_copy(r0, acc.at[pl.ds(s * RPT + k * BLK, BLK)])
    plsc.subcore_barrier()

    ebase = wid * NBLK

    def issue_gather(k, p):
        pltpu.async_copy(h_hbm.at[rowb.at[k]], bufs[p], gsem[p])

    def wait_gather(k, p):
        pltpu.make_async_copy(h_hbm.at[rowb.at[k]], bufs[p], gsem[p]).wait()

    def issue_scatter(k, p):
        pltpu.async_copy(bufs[p], acc.at[colb.at[k]], ssem[p], add=True)

    def wait_scatter(k, p):
        pltpu.make_async_copy(bufs[p], acc.at[colb.at[k]], ssem[p]).wait()

    def scale_buf(buf, k):
        def scale(g, carry):
            wv = wb[k, pl.ds(g * 16, 16)]
            e0 = g * 16
            for l in range(16):
                sv = wv[l]
                for i in range(8):
                    buf[e0 + l, pl.ds(i * 16, 16)] = (
                        buf[e0 + l, pl.ds(i * 16, 16)] * sv)
            return carry

        lax.fori_loop(0, BLK // 16, scale, 0)

    for t in range(2):
        if t > 0:
            wait_scatter(CH - 1, (CH - 1) % 2)
        pltpu.sync_copy(row_hbm.at[pl.ds(ebase + t * CH, CH)], rowb)
        pltpu.sync_copy(col_hbm.at[pl.ds(ebase + t * CH, CH)], colb)
        pltpu.sync_copy(w_hbm.at[pl.ds(ebase + t * CH, CH)], wb)
        issue_gather(0, 0)
        issue_gather(1, 1)
        wait_gather(0, 0)
        scale_buf(bufs[0], 0)
        issue_scatter(0, 0)

        def pair(u, carry):
            k0 = 1 + 2 * u
            for par, k_off in ((1, 0), (0, 1)):
                k = k0 + k_off
                wait_scatter(k - 1, 1 - par)
                issue_gather(k + 1, 1 - par)
                wait_gather(k, par)
                scale_buf(bufs[par], k)
                issue_scatter(k, par)
            return carry

        lax.fori_loop(0, (CH - 2) // 2, pair, 0)

        k = CH - 1
        wait_scatter(k - 1, k % 2 ^ 1)
        wait_gather(k, k % 2)
        scale_buf(bufs[k % 2], k)
        issue_scatter(k, k % 2)

    wait_scatter(CH - 1, (CH - 1) % 2)
    plsc.subcore_barrier()
    pltpu.sync_copy(acc.at[pl.ds(s * RPT, RPT)],
                    out_hbm.at[c, pl.ds(s * RPT, RPT)])


def _prep_body(x_ref, dp_ref, h0t_ref, dinv_ref):
    deg = 1.0 + dp_ref[:, 0:1] + dp_ref[:, 1:2]
    dinv = jnp.where(deg > 0, lax.rsqrt(deg), 0.0)
    h0t_ref[...] = x_ref[...] * dinv
    dinv_ref[...] = dinv


_TCB = 1000


def _prep_call(x, dp):
    return pl.pallas_call(
        _prep_body,
        grid=(N // _TCB,),
        in_specs=[
            pl.BlockSpec((_TCB, D), lambda i: (i, 0)),
            pl.BlockSpec((_TCB, 2), lambda i: (i, 0)),
        ],
        out_specs=[
            pl.BlockSpec((_TCB, D), lambda i: (i, 0)),
            pl.BlockSpec((_TCB, 1), lambda i: (i, 0)),
        ],
        out_shape=[
            jax.ShapeDtypeStruct((N, D), jnp.float32),
            jax.ShapeDtypeStruct((N, 1), jnp.float32),
        ],
    )(x, dp)


def _mid_body(p_ref, h0t_ref, dinv_ref, out_ref):
    t = p_ref[0] + p_ref[1] + h0t_ref[...]
    d = dinv_ref[...]
    out_ref[...] = t * (d * d)


def _mid_call(p, h0t, dinv):
    return pl.pallas_call(
        _mid_body,
        grid=(N // _TCB,),
        in_specs=[
            pl.BlockSpec((2, _TCB, D), lambda i: (0, i, 0)),
            pl.BlockSpec((_TCB, D), lambda i: (i, 0)),
            pl.BlockSpec((_TCB, 1), lambda i: (i, 0)),
        ],
        out_specs=pl.BlockSpec((_TCB, D), lambda i: (i, 0)),
        out_shape=jax.ShapeDtypeStruct((N, D), jnp.float32),
    )(p, h0t, dinv)


def _final_body(q_ref, h1t_ref, dinv_ref, w_ref, b_ref, out_ref):
    h2 = (q_ref[0] + q_ref[1] + h1t_ref[...]) * dinv_ref[...]
    z = jnp.dot(h2, w_ref[...], preferred_element_type=jnp.float32) + b_ref[...]
    z = jnp.maximum(z, 0.0)
    m = jnp.max(z, axis=-1, keepdims=True)
    lse = jnp.log(jnp.sum(jnp.exp(z - m), axis=-1, keepdims=True)) + m
    out_ref[...] = z - lse


def _final_call(q, h1t, dinv, w, b2):
    return pl.pallas_call(
        _final_body,
        grid=(N // _TCB,),
        in_specs=[
            pl.BlockSpec((2, _TCB, D), lambda i: (0, i, 0)),
            pl.BlockSpec((_TCB, D), lambda i: (i, 0)),
            pl.BlockSpec((_TCB, 1), lambda i: (i, 0)),
            pl.BlockSpec((D, D), lambda i: (0, 0)),
            pl.BlockSpec((1, D), lambda i: (0, 0)),
        ],
        out_specs=pl.BlockSpec((_TCB, D), lambda i: (i, 0)),
        out_shape=jax.ShapeDtypeStruct((N, D), jnp.float32),
    )(q, h1t, dinv, w, b2)


def kernel(x, edge_index, edge_attr, W, b):
    pad_ar = jnp.arange(E_PAD - E, dtype=jnp.int32)
    row_pad = pad_ar % N
    col_pad = N + pad_ar % (N_PAD - N)
    row = jnp.concatenate([edge_index[0], row_pad]).reshape(E_PAD // BLK, BLK)
    col = jnp.concatenate([edge_index[1], col_pad]).reshape(E_PAD // BLK, BLK)
    w2 = jnp.concatenate(
        [edge_attr, jnp.zeros((E_PAD - E,), jnp.float32)]).reshape(E_PAD // BLK, BLK)

    degp = _deg_kernel(col, w2)
    dp = jnp.transpose(degp[:, :N])
    h0t, dinv = _prep_call(x, dp)
    p = _prop_kernel(h0t, row, col, w2)
    h1t = _mid_call(p, h0t, dinv)
    q = _prop_kernel(h1t, row, col, w2)
    return _final_call(q, h1t, dinv, W, b.reshape(1, D))
